# baseline (speedup 1.0000x reference)
import functools

import jax
import jax.numpy as jnp
from jax import lax
from jax.experimental import pallas as pl
from jax.experimental.pallas import tpu as pltpu
from jax.experimental.pallas import tpu_sc as plsc

NC = 2
NS = 16
NW = NC * NS
L = 16


@functools.lru_cache(maxsize=None)
def _build(V, K, C, F, B):
    assert F == L and V % B == 0 and K == L and B % 8 == 0
    W = (F + C + 7) // 8 * 8
    CHUNKS = V // B
    OUTW = F * C * C + F * C
    GSZ = 128
    assert (B * K) % GSZ == 0
    GROUPS = B * K // GSZ
    MYI = (CHUNKS + NW - 1) // NW
    NPAIRS = (MYI + 1) // 2
    mesh = plsc.VectorSubcoreMesh(
        core_axis_name="c", subcore_axis_name="s", num_cores=NC, num_subcores=NS
    )

    @functools.partial(
        pl.kernel,
        out_type=jax.ShapeDtypeStruct((OUTW, V), jnp.float32),
        mesh=mesh,
        scratch_types=[
            pltpu.VMEM((GROUPS, GSZ), jnp.int32),
            pltpu.VMEM((GROUPS, GSZ), jnp.int32),
            pltpu.VMEM((B, K), jnp.float32),
            pltpu.VMEM((B, K), jnp.float32),
            pltpu.VMEM((B * K, W), jnp.float32),
            pltpu.VMEM((B * K, W), jnp.float32),
            pltpu.VMEM((OUTW, B), jnp.float32),
            pltpu.VMEM((OUTW, B), jnp.float32),
            pltpu.SemaphoreType.DMA,
            pltpu.SemaphoreType.DMA,
            pltpu.SemaphoreType.DMA,
            pltpu.SemaphoreType.DMA,
        ],
        compiler_params=pltpu.CompilerParams(
            use_tc_tiling_on_sc=False, needs_layout_passes=False),
    )
    def nbcov(tbl_hbm, dsq_hbm, idx_hbm, out_hbm,
              idx0, idx1, dsq0, dsq1, rows0, rows1, out0, out1,
              semg0, semg1, semo0, semo1):
        wid = lax.axis_index("s") * NC + lax.axis_index("c")
        iota = lax.iota(jnp.int32, L)
        base_cov = iota * (C * C)
        base_mean = iota * C + F * C * C
        idx_v = (idx0, idx1)
        dsq_v = (dsq0, dsq1)
        rows_v = (rows0, rows1)
        out_v = (out0, out1)
        semg = (semg0, semg1)
        semo = (semo0, semo1)

        def stage_inputs(chunk, s):
            pltpu.sync_copy(idx_hbm.at[pl.ds(chunk * GROUPS, GROUPS)], idx_v[s])
            pltpu.sync_copy(dsq_hbm.at[pl.ds(chunk * B, B)], dsq_v[s])
            for g in range(GROUPS):
                pltpu.async_copy(tbl_hbm.at[idx_v[s].at[g]],
                                 rows_v[s].at[pl.ds(g * GSZ, GSZ)], semg[s])

        def drain_gathers(s):
            for g in range(GROUPS):
                pltpu.make_async_copy(tbl_hbm.at[idx_v[s].at[g]],
                                      rows_v[s].at[pl.ds(g * GSZ, GSZ)],
                                      semg[s]).wait()

        def drain_out(s):
            pltpu.make_async_copy(
                out_hbm.at[:, pl.ds(0, B)], out_v[s], semo[s]).wait()

        def compute_chunk(s):
            rows = rows_v[s]
            dsq = dsq_v[s]
            out = out_v[s]

            @plsc.parallel_loop(0, B, 1, unroll=1)
            def vertex(b):
                row0 = b * K
                bfull = jnp.full((L,), b, jnp.int32)
                w = jnp.exp(dsq[b, :] * -10.0)
                cvec = [plsc.load_gather(rows, [row0 + iota,
                                                jnp.full((L,), F + c, jnp.int32)])
                        for c in range(C)]
                sumw = jnp.zeros((L,), jnp.float32)
                m = [jnp.zeros((L,), jnp.float32) for _ in range(C)]
                e = {(c, d): jnp.zeros((L,), jnp.float32)
                     for c in range(C) for d in range(c, C)}
                for k in range(K):
                    kfull = jnp.full((L,), k, jnp.int32)
                    frow = rows[row0 + k, pl.ds(0, F)]
                    wb = jnp.take_along_axis(w, kfull, axis=0,
                                             mode="promise_in_bounds")
                    cb = [jnp.take_along_axis(cvec[c], kfull, axis=0,
                                              mode="promise_in_bounds")
                          for c in range(C)]
                    wf = frow * wb
                    sumw = sumw + wf
                    for c in range(C):
                        t = wf * cb[c]
                        m[c] = m[c] + t
                        for d in range(c, C):
                            e[(c, d)] = e[(c, d)] + t * cb[d]
                iszero = sumw == 0.0
                r_ = jnp.where(iszero, 0.0, 1.0 / sumw)
                mm = [m[c] * r_ for c in range(C)]
                for c in range(C):
                    plsc.store_scatter(out, [base_mean + c, bfull], mm[c])
                    for d in range(c, C):
                        cov = e[(c, d)] * r_ - mm[c] * mm[d]
                        plsc.store_scatter(out, [base_cov + (c * C + d), bfull], cov)
                        if d > c:
                            plsc.store_scatter(out, [base_cov + (d * C + c), bfull], cov)

        def half(p, s):
            i = 2 * p + s
            ci = wid + i * NW
            cn = ci + 2 * NW

            def work():
                drain_gathers(s)
                pl.when(i >= 2)(lambda: drain_out(s))
                compute_chunk(s)
                pltpu.async_copy(out_v[s], out_hbm.at[:, pl.ds(ci * B, B)],
                                 semo[s])
                pl.when(cn < CHUNKS)(lambda: stage_inputs(cn, s))

            pl.when(ci < CHUNKS)(work)

        for s in range(2):
            c0 = wid + s * NW
            pl.when(c0 < CHUNKS)(lambda s=s, c0=c0: stage_inputs(c0, s))

        def pair(p, _):
            half(p, 0)
            half(p, 1)
            return 0

        lax.fori_loop(0, NPAIRS, pair, 0, unroll=False)

        my_n = (CHUNKS - wid + NW - 1) // NW
        pl.when(my_n >= 1)(lambda: drain_out(0))
        pl.when(my_n >= 2)(lambda: drain_out(1))

    return nbcov


def kernel(coordinates, distsq, features, n_idxs):
    V, C = coordinates.shape
    _, K = n_idxs.shape
    _, F = features.shape
    B = 80
    W = (F + C + 7) // 8 * 8
    tbl = jnp.concatenate(
        [features, coordinates,
         jnp.zeros((V, W - F - C), jnp.float32)], axis=1)
    fn = _build(V, K, C, F, B)
    return fn(tbl, distsq, n_idxs.reshape(-1, 128)).T


# speedup vs baseline: 1.5087x; 1.5087x over previous
import functools

import jax
import jax.numpy as jnp
from jax import lax
from jax.experimental import pallas as pl
from jax.experimental.pallas import tpu as pltpu
from jax.experimental.pallas import tpu_sc as plsc

NC = 2
NS = 16
NW = NC * NS
L = 16


@functools.lru_cache(maxsize=None)
def _build(V, K, C, F, B):
    assert F == L and V % B == 0 and K == L and B % 8 == 0
    W = (F + C + 7) // 8 * 8
    CHUNKS = V // B
    OUTW = F * C * C + F * C
    GSZ = 128
    assert (B * K) % GSZ == 0
    GROUPS = B * K // GSZ
    MYI = (CHUNKS + NW - 1) // NW
    NPAIRS = (MYI + 1) // 2
    mesh = plsc.VectorSubcoreMesh(
        core_axis_name="c", subcore_axis_name="s", num_cores=NC, num_subcores=NS
    )

    @functools.partial(
        pl.kernel,
        out_type=jax.ShapeDtypeStruct((OUTW, V), jnp.float32),
        mesh=mesh,
        scratch_types=[
            pltpu.VMEM((GROUPS, GSZ), jnp.int32),
            pltpu.VMEM((GROUPS, GSZ), jnp.int32),
            pltpu.VMEM((B, K), jnp.float32),
            pltpu.VMEM((B, K), jnp.float32),
            pltpu.VMEM((B * K, W), jnp.float32),
            pltpu.VMEM((B * K, W), jnp.float32),
            pltpu.VMEM((OUTW, B), jnp.float32),
            pltpu.VMEM((OUTW, B), jnp.float32),
            pltpu.SemaphoreType.DMA,
            pltpu.SemaphoreType.DMA,
            pltpu.SemaphoreType.DMA,
            pltpu.SemaphoreType.DMA,
        ],
        compiler_params=pltpu.CompilerParams(
            use_tc_tiling_on_sc=False, needs_layout_passes=False),
    )
    def nbcov(tbl_hbm, dsq_hbm, idx_hbm, out_hbm,
              idx0, idx1, dsq0, dsq1, rows0, rows1, out0, out1,
              semg0, semg1, semo0, semo1):
        wid = lax.axis_index("s") * NC + lax.axis_index("c")
        iota = lax.iota(jnp.int32, L)
        base_cov = iota * (C * C)
        base_mean = iota * C + F * C * C
        idx_v = (idx0, idx1)
        dsq_v = (dsq0, dsq1)
        rows_v = (rows0, rows1)
        out_v = (out0, out1)
        semg = (semg0, semg1)
        semo = (semo0, semo1)

        def stage_inputs(chunk, s):
            pltpu.sync_copy(idx_hbm.at[pl.ds(chunk * GROUPS, GROUPS)], idx_v[s])
            pltpu.sync_copy(dsq_hbm.at[pl.ds(chunk * B, B)], dsq_v[s])
            for g in range(GROUPS):
                pltpu.async_copy(tbl_hbm.at[idx_v[s].at[g]],
                                 rows_v[s].at[pl.ds(g * GSZ, GSZ)], semg[s])

        def drain_gathers(s):
            for g in range(GROUPS):
                pltpu.make_async_copy(tbl_hbm.at[idx_v[s].at[g]],
                                      rows_v[s].at[pl.ds(g * GSZ, GSZ)],
                                      semg[s]).wait()

        def drain_out(s):
            pltpu.make_async_copy(
                out_hbm.at[:, pl.ds(0, B)], out_v[s], semo[s]).wait()

        def compute_chunk(s):
            rows = rows_v[s]
            dsq = dsq_v[s]
            out = out_v[s]

            def vertex(b, _):
                row0 = b * K
                bfull = jnp.full((L,), b, jnp.int32)
                w = jnp.exp(dsq[b, :] * -10.0)
                cvec = [plsc.load_gather(rows, [row0 + iota,
                                                jnp.full((L,), F + c, jnp.int32)])
                        for c in range(C)]
                sumw = jnp.zeros((L,), jnp.float32)
                m = [jnp.zeros((L,), jnp.float32) for _ in range(C)]
                e = {(c, d): jnp.zeros((L,), jnp.float32)
                     for c in range(C) for d in range(c, C)}
                for k in range(K):
                    kfull = jnp.full((L,), k, jnp.int32)
                    frow = rows[row0 + k, pl.ds(0, F)]
                    wb = jnp.take_along_axis(w, kfull, axis=0,
                                             mode="promise_in_bounds")
                    cb = [jnp.take_along_axis(cvec[c], kfull, axis=0,
                                              mode="promise_in_bounds")
                          for c in range(C)]
                    wf = frow * wb
                    sumw = sumw + wf
                    for c in range(C):
                        t = wf * cb[c]
                        m[c] = m[c] + t
                        for d in range(c, C):
                            e[(c, d)] = e[(c, d)] + t * cb[d]
                iszero = sumw == 0.0
                r_ = jnp.where(iszero, 0.0, 1.0 / sumw)
                mm = [m[c] * r_ for c in range(C)]
                for c in range(C):
                    plsc.store_scatter(out, [base_mean + c, bfull], mm[c])
                    for d in range(c, C):
                        cov = e[(c, d)] * r_ - mm[c] * mm[d]
                        plsc.store_scatter(out, [base_cov + (c * C + d), bfull], cov)
                        if d > c:
                            plsc.store_scatter(out, [base_cov + (d * C + c), bfull], cov)
                return 0

            lax.fori_loop(0, B, vertex, 0, unroll=False)

        def half(p, s):
            i = 2 * p + s
            ci = wid + i * NW
            cn = ci + 2 * NW

            def work():
                drain_gathers(s)
                pl.when(i >= 2)(lambda: drain_out(s))
                compute_chunk(s)
                pltpu.async_copy(out_v[s], out_hbm.at[:, pl.ds(ci * B, B)],
                                 semo[s])
                pl.when(cn < CHUNKS)(lambda: stage_inputs(cn, s))

            pl.when(ci < CHUNKS)(work)

        for s in range(2):
            c0 = wid + s * NW
            pl.when(c0 < CHUNKS)(lambda s=s, c0=c0: stage_inputs(c0, s))

        def pair(p, _):
            half(p, 0)
            half(p, 1)
            return 0

        lax.fori_loop(0, NPAIRS, pair, 0, unroll=False)

        my_n = (CHUNKS - wid + NW - 1) // NW
        pl.when(my_n >= 1)(lambda: drain_out(0))
        pl.when(my_n >= 2)(lambda: drain_out(1))

    return nbcov


def kernel(coordinates, distsq, features, n_idxs):
    V, C = coordinates.shape
    _, K = n_idxs.shape
    _, F = features.shape
    B = 80
    W = (F + C + 7) // 8 * 8
    tbl = jnp.concatenate(
        [features, coordinates,
         jnp.zeros((V, W - F - C), jnp.float32)], axis=1)
    fn = _build(V, K, C, F, B)
    return fn(tbl, distsq, n_idxs.reshape(-1, 128)).T
